# baseline (speedup 1.0000x reference)
import functools

import jax
import jax.numpy as jnp
from jax import lax
from jax.experimental import pallas as pl
from jax.experimental.pallas import tpu as pltpu
from jax.experimental.pallas import tpu_sc as plsc

_SUB = 128
_NB = 5


@functools.partial(jax.jit, static_argnames=("n_rows",))
def _sc_embed(idx_grouped, table_padded, n_rows):
    info = plsc.get_sparse_core_info()
    nc, ns = info.num_cores, info.num_subcores
    nw = nc * ns
    b_per_w = n_rows // nw
    n_sub = b_per_w // _SUB
    two_d = table_padded.shape[1]

    mesh = plsc.VectorSubcoreMesh(core_axis_name="c", subcore_axis_name="s")

    @functools.partial(
        pl.kernel,
        out_type=jax.ShapeDtypeStruct((n_rows, two_d), jnp.float32),
        mesh=mesh,
        scratch_types=[
            pltpu.VMEM((n_sub, _SUB), jnp.int32),
            [pltpu.VMEM((_SUB, two_d), jnp.float32)] * _NB,
            [pltpu.SemaphoreType.DMA] * _NB,
            [pltpu.SemaphoreType.DMA] * _NB,
        ],
        compiler_params=pltpu.CompilerParams(
            use_tc_tiling_on_sc=True, needs_layout_passes=False
        ),
    )
    def body(idx_hbm, tbl_hbm, out_hbm, idx_v, stages, gsems, wsems):
        wid = lax.axis_index("s") * nc + lax.axis_index("c")
        base = wid * b_per_w
        pltpu.sync_copy(idx_hbm.at[wid], idx_v)

        def fire(j, b):
            pltpu.async_copy(tbl_hbm.at[idx_v.at[j]], stages[b], gsems[b])

        def drain(j, b):
            pltpu.make_async_copy(
                tbl_hbm.at[idx_v.at[j]], stages[b], gsems[b]
            ).wait()
            pltpu.async_copy(
                stages[b], out_hbm.at[pl.ds(base + j * _SUB, _SUB)], wsems[b]
            )

        def wait_write(j, b):
            pltpu.make_async_copy(
                stages[b], out_hbm.at[pl.ds(base + j * _SUB, _SUB)], wsems[b]
            ).wait()

        for b in range(_NB):
            fire(b, b)

        def step(t, carry):
            for b in range(_NB):
                j = t * _NB + b
                drain(j, b)
                nxt = j + _NB

                @pl.when(nxt < n_sub)
                def _refill():
                    wait_write(j, b)
                    fire(nxt, b)

            return carry

        lax.fori_loop(0, n_sub // _NB, step, 0)
        for b in range(_NB):
            wait_write(n_sub - _NB + b, b)

    return body(idx_grouped, table_padded)


_PB = 8000


@jax.jit
def _tc_widen(table):
    vocab, emb = table.shape

    def body(t_ref, o_ref):
        o_ref[:, :emb] = t_ref[...]

    return pl.pallas_call(
        body,
        grid=(vocab // _PB,),
        in_specs=[pl.BlockSpec((_PB, emb), lambda i: (i, 0))],
        out_specs=pl.BlockSpec((_PB, 2 * emb), lambda i: (i, 0)),
        out_shape=jax.ShapeDtypeStruct((vocab, 2 * emb), jnp.float32),
    )(table)


def kernel(input_ids, embedding_weight):
    batch, hist = input_ids.shape
    vocab, embed_dim = embedding_weight.shape
    n_rows = batch * hist

    info = plsc.get_sparse_core_info()
    nw = info.num_cores * info.num_subcores
    b_per_w = n_rows // nw

    table_padded = _tc_widen(embedding_weight)
    idx_grouped = input_ids.astype(jnp.int32).reshape(nw, b_per_w // _SUB, _SUB)
    out = _sc_embed(idx_grouped, table_padded, n_rows)
    return out[:, :embed_dim].reshape(batch, hist, embed_dim)


# speedup vs baseline: 1.1649x; 1.1649x over previous
import functools

import jax
import jax.numpy as jnp
from jax import lax
from jax.experimental import pallas as pl
from jax.experimental.pallas import tpu as pltpu
from jax.experimental.pallas import tpu_sc as plsc

_SUB = 128
_NB = 5


@functools.partial(jax.jit, static_argnames=("n_rows",))
def _sc_embed(idx_grouped, table_padded, n_rows):
    info = plsc.get_sparse_core_info()
    nc, ns = info.num_cores, info.num_subcores
    nw = nc * ns
    b_per_w = n_rows // nw
    n_sub = b_per_w // _SUB
    two_d = table_padded.shape[1]

    mesh = plsc.VectorSubcoreMesh(core_axis_name="c", subcore_axis_name="s")

    @functools.partial(
        pl.kernel,
        out_type=jax.ShapeDtypeStruct((n_rows, two_d), jnp.float32),
        mesh=mesh,
        scratch_types=[
            pltpu.VMEM((n_sub, _SUB), jnp.int32),
            [pltpu.VMEM((_SUB, two_d), jnp.float32)] * _NB,
            [pltpu.SemaphoreType.DMA] * _NB,
            [pltpu.SemaphoreType.DMA] * _NB,
        ],
        compiler_params=pltpu.CompilerParams(
            use_tc_tiling_on_sc=True, needs_layout_passes=False
        ),
    )
    def body(idx_hbm, tbl_hbm, out_hbm, idx_v, stages, gsems, wsems):
        wid = lax.axis_index("s") * nc + lax.axis_index("c")
        base = wid * b_per_w
        pltpu.sync_copy(idx_hbm.at[wid], idx_v)

        def fire(j, b):
            pltpu.async_copy(tbl_hbm.at[idx_v.at[j]], stages[b], gsems[b])

        def drain(j, b):
            pltpu.make_async_copy(
                tbl_hbm.at[idx_v.at[j]], stages[b], gsems[b]
            ).wait()
            pltpu.async_copy(
                stages[b], out_hbm.at[pl.ds(base + j * _SUB, _SUB)], wsems[b]
            )

        def wait_write(j, b):
            pltpu.make_async_copy(
                stages[b], out_hbm.at[pl.ds(base + j * _SUB, _SUB)], wsems[b]
            ).wait()

        for b in range(_NB):
            fire(b, b)

        def step(t, carry):
            for b in range(_NB):
                j = t * _NB + b
                drain(j, b)
                nxt = j + _NB

                @pl.when(nxt < n_sub)
                def _refill():
                    wait_write(j, b)
                    fire(nxt, b)

            return carry

        lax.fori_loop(0, n_sub // _NB, step, 0)
        for b in range(_NB):
            wait_write(n_sub - _NB + b, b)

    return body(idx_grouped, table_padded)


def kernel(input_ids, embedding_weight):
    batch, hist = input_ids.shape
    vocab, embed_dim = embedding_weight.shape
    n_rows = batch * hist

    info = plsc.get_sparse_core_info()
    nw = info.num_cores * info.num_subcores
    b_per_w = n_rows // nw

    table_padded = jnp.pad(embedding_weight, ((0, 0), (0, 128 - embed_dim)))
    idx_grouped = input_ids.astype(jnp.int32).reshape(nw, b_per_w // _SUB, _SUB)
    out = _sc_embed(idx_grouped, table_padded, n_rows)
    return out[:, :embed_dim].reshape(batch, hist, embed_dim)
